# baseline (speedup 1.0000x reference)
import functools

import jax
import jax.numpy as jnp
from jax import lax
from jax.experimental import pallas as pl
from jax.experimental.pallas import tpu as pltpu
from jax.experimental.pallas import tpu_sc as plsc

_V = 100000
_D = 128
_C = 2048
_B = 512
_WORDS = 6
_CPW = 5
_IMG = 38
_NEG = 5
_MINI = _B // 2

_NC = 2
_NS = 16
_NW = _NC * _NS


def _sc_gather_call(table, idx):
    B, = idx.shape
    V, Dg = table.shape
    b_per_w = B // _NW
    max_rows = max(8, (360 * 1024) // (Dg * 4))
    chunk = b_per_w
    while chunk > max_rows or chunk % 8 != 0:
        chunk //= 2
    n_chunks = b_per_w // chunk
    mesh = plsc.VectorSubcoreMesh(core_axis_name="c", subcore_axis_name="s",
                                  num_cores=_NC, num_subcores=_NS)

    @functools.partial(
        pl.kernel, mesh=mesh,
        out_type=jax.ShapeDtypeStruct((B, Dg), table.dtype),
        scratch_types=[
            pltpu.VMEM((chunk,), jnp.int32),
            pltpu.VMEM((chunk, Dg), table.dtype),
            pltpu.SemaphoreType.DMA,
        ],
    )
    def k(table_hbm, idx_hbm, out_hbm, idx_v, rows_v, sem):
        wid = lax.axis_index("s") * _NC + lax.axis_index("c")
        base = wid * b_per_w

        def body(ci, carry):
            off = base + ci * chunk
            pltpu.sync_copy(idx_hbm.at[pl.ds(off, chunk)], idx_v)
            pltpu.async_copy(table_hbm.at[idx_v], rows_v, sem).wait()
            pltpu.sync_copy(rows_v, out_hbm.at[pl.ds(off, chunk)])
            return carry

        lax.fori_loop(0, n_chunks, body, 0)

    return k(table, idx)


def _gather_rows(table, idx):
    return _sc_gather_call(table, idx)


_R = 800


def _cdf_kernel(p2_ref, cdf_ref, coarse_ref):
    p2 = p2_ref[...]
    total = jnp.sum(p2)
    p2 = p2 / total
    li = lax.broadcasted_iota(jnp.int32, (128, 128), 0)
    lj = lax.broadcasted_iota(jnp.int32, (128, 128), 1)
    lt = (li <= lj).astype(jnp.float32)
    rowcum = jnp.dot(p2, lt, preferred_element_type=jnp.float32)
    rs = jnp.dot(p2, jnp.ones((128, 1), jnp.float32),
                 preferred_element_type=jnp.float32)
    ri = lax.broadcasted_iota(jnp.int32, (_R, _R), 0)
    rj = lax.broadcasted_iota(jnp.int32, (_R, _R), 1)
    mstrict = (rj < ri).astype(jnp.float32)
    excl = jnp.dot(mstrict, rs, preferred_element_type=jnp.float32)
    cdf = rowcum + excl
    cdf_ref[...] = cdf
    e127 = (lax.broadcasted_iota(jnp.int32, (1, 128), 1) == 127
            ).astype(jnp.float32)
    coarse_ref[...] = lax.dot_general(
        e127, cdf, (((1,), (1,)), ((), ())),
        preferred_element_type=jnp.float32)


def _build_cdf(noise_dist):
    p_pad = jnp.concatenate(
        [noise_dist, jnp.zeros((_R * 128 - _V,), jnp.float32)]).reshape(_R, 128)
    return pl.pallas_call(
        _cdf_kernel,
        out_shape=(jax.ShapeDtypeStruct((_R, 128), jnp.float32),
                   jax.ShapeDtypeStruct((1, _R), jnp.float32)),
    )(p_pad)


_UBLK = 1024


def _ss_kernel(u_ref, coarse_ref, cdf_ref, out_ref):
    u = u_ref[...]
    coarse = coarse_ref[...]
    ind1 = (coarse < u).astype(jnp.float32)
    count1 = jnp.sum(ind1, axis=1, keepdims=True)
    rlane = lax.broadcasted_iota(jnp.int32, (_UBLK, _R), 1)
    onehot = (rlane == count1.astype(jnp.int32)).astype(jnp.float32)
    seg = jnp.dot(onehot, cdf_ref[...],
                  preferred_element_type=jnp.float32)
    count2 = jnp.sum((seg < u).astype(jnp.float32), axis=1, keepdims=True)
    idx = count1 * 128.0 + count2
    idx = jnp.clip(idx, 0.0, float(_V - 1))
    out_ref[...] = idx.astype(jnp.int32)


def _searchsorted(uu_flat, cdf, coarse):
    n = uu_flat.shape[0]
    grid = n // _UBLK
    return pl.pallas_call(
        _ss_kernel,
        grid=(grid,),
        in_specs=[
            pl.BlockSpec((_UBLK, 1), lambda i: (i, 0)),
            pl.BlockSpec((1, _R), lambda i: (0, 0)),
            pl.BlockSpec((_R, 128), lambda i: (0, 0)),
        ],
        out_specs=pl.BlockSpec((_UBLK, 1), lambda i: (i, 0)),
        out_shape=jax.ShapeDtypeStruct((n, 1), jnp.int32),
    )(uu_flat.reshape(n, 1), coarse, cdf)


_SB = 64


def _sims_kernel(v6_ref, nv_ref, words_ref, avg_ref, nemb_ref, nsamp_ref):
    v6 = v6_ref[...]
    avg = jnp.sum(v6, axis=1) / float(_WORDS)
    avg_ref[...] = avg
    nv = nv_ref[...]
    num = jnp.sum(nv * avg[:, None, :], axis=2)
    na = jnp.sqrt(jnp.sum(avg * avg, axis=1, keepdims=True))
    nn = jnp.sqrt(jnp.sum(nv * nv, axis=2))
    den = jnp.maximum(na * nn, 1e-8)
    sims = num / den
    words = words_ref[...]
    miota = lax.broadcasted_iota(jnp.int32, (_SB, _MINI), 1)
    cur = sims
    samp_cols = []
    for k in range(_NEG):
        m = jnp.max(cur, axis=1, keepdims=True)
        cand = jnp.where(cur >= m, miota, _MINI + 1)
        idxk = jnp.min(cand, axis=1, keepdims=True)
        onehot = (miota == idxk).astype(jnp.float32)
        nemb_ref[:, k, :] = jnp.sum(nv * onehot[:, :, None], axis=1)
        samp_cols.append(jnp.sum(words * onehot, axis=1, keepdims=True))
        cur = jnp.where(miota == idxk, -jnp.inf, cur)
    nsamp_ref[...] = jnp.concatenate(
        samp_cols + [jnp.zeros((_SB, 3), jnp.float32)], axis=1)


def _sims_topk(v_rows6, noise_vec, words_f):
    grid = _B // _SB
    return pl.pallas_call(
        _sims_kernel,
        grid=(grid,),
        in_specs=[
            pl.BlockSpec((_SB, _WORDS, 128), lambda i: (i, 0, 0)),
            pl.BlockSpec((_SB, _MINI, 128), lambda i: (i, 0, 0)),
            pl.BlockSpec((_SB, _MINI), lambda i: (i, 0)),
        ],
        out_specs=(
            pl.BlockSpec((_SB, 128), lambda i: (i, 0)),
            pl.BlockSpec((_SB, _NEG, 128), lambda i: (i, 0, 0)),
            pl.BlockSpec((_SB, 8), lambda i: (i, 0)),
        ),
        out_shape=(
            jax.ShapeDtypeStruct((_B, 128), jnp.float32),
            jax.ShapeDtypeStruct((_B, _NEG, 128), jnp.float32),
            jax.ShapeDtypeStruct((_B, 8), jnp.float32),
        ),
    )(v_rows6, noise_vec, words_f)


_NB1 = 32
_NB2 = 64


def _lane_roll32(x):
    return jnp.maximum(x, jnp.concatenate([x[:, 32:], x[:, :32]], axis=1))


def _conv1_kernel(img_ref, m1_ref, sel1_ref, out_ref, sum_ref, sq_ref):
    acc = jnp.zeros((_NB1 * 36, 1152), jnp.float32)
    for dy in range(3):
        s = img_ref[:, dy:dy + 36, :].reshape(_NB1 * 36, 38)
        acc = acc + jnp.dot(s, m1_ref[dy * 38:(dy + 1) * 38, :],
                            preferred_element_type=jnp.float32)

    @pl.when(pl.program_id(0) == 0)
    def _():
        sum_ref[...] = jnp.zeros_like(sum_ref)
        sq_ref[...] = jnp.zeros_like(sq_ref)

    sum_ref[...] += jnp.sum(acc, axis=0, keepdims=True)
    sq_ref[...] += jnp.sum(acc * acc, axis=0, keepdims=True)
    a = jnp.max(acc.reshape(_NB1, 18, 2, 1152), axis=2).reshape(_NB1 * 18, 1152)
    m = _lane_roll32(a)
    out_ref[...] = jnp.dot(m, sel1_ref[...],
                           preferred_element_type=jnp.float32
                           ).reshape(_NB1, 18, 576)


def _conv2_kernel(x_ref, m2_ref, sel2_ref, f5_ref, sum1_ref, sq1_ref,
                  f1_ref, g1_ref, out_ref, sum_ref, sq_ref):
    n1 = float(_C * 36 * 36)
    mean1 = jnp.dot(sum1_ref[...], f1_ref[...],
                    preferred_element_type=jnp.float32) / n1
    var1 = jnp.dot(sq1_ref[...], f1_ref[...],
                   preferred_element_type=jnp.float32) / n1 - mean1 * mean1
    a1 = g1_ref[...] / jnp.sqrt(var1 + 1e-5)
    a1col = lax.dot_general(f5_ref[...], a1, (((1,), (1,)), ((), ())),
                            preferred_element_type=jnp.float32)
    acc = jnp.zeros((_NB2 * 16, 512), jnp.float32)
    for dy in range(3):
        s = x_ref[:, dy:dy + 16, :].reshape(_NB2 * 16, 576)
        m2 = m2_ref[dy * 576:(dy + 1) * 576, :] * a1col
        acc = acc + jnp.dot(s, m2, preferred_element_type=jnp.float32)

    @pl.when(pl.program_id(0) == 0)
    def _():
        sum_ref[...] = jnp.zeros_like(sum_ref)
        sq_ref[...] = jnp.zeros_like(sq_ref)

    sum_ref[...] += jnp.sum(acc, axis=0, keepdims=True)
    sq_ref[...] += jnp.sum(acc * acc, axis=0, keepdims=True)
    a = jnp.max(acc.reshape(_NB2, 8, 2, 512), axis=2).reshape(_NB2 * 8, 512)
    m = _lane_roll32(a)
    out_ref[...] = jnp.dot(m, sel2_ref[...],
                           preferred_element_type=jnp.float32
                           ).reshape(_NB2, 8, 256)


def _fc_bn3_kernel(x_ref, w_ref, ffc_ref, sum2_ref, sq2_ref, f2_ref, g2_ref,
                   g3_ref, b3_ref, out_ref):
    n2 = float(_C * 16 * 16)
    mean2 = jnp.dot(sum2_ref[...], f2_ref[...],
                    preferred_element_type=jnp.float32) / n2
    var2 = jnp.dot(sq2_ref[...], f2_ref[...],
                   preferred_element_type=jnp.float32) / n2 - mean2 * mean2
    a2 = g2_ref[...] / jnp.sqrt(var2 + 1e-5)
    a2col = lax.dot_general(ffc_ref[...], a2, (((1,), (1,)), ((), ())),
                            preferred_element_type=jnp.float32)
    y = jnp.dot(x_ref[...], w_ref[...] * a2col,
                preferred_element_type=jnp.float32)
    m = jnp.mean(y, axis=0, keepdims=True)
    d = y - m
    var = jnp.mean(d * d, axis=0, keepdims=True)
    z = g3_ref[...] * d / jnp.sqrt(var + 1e-5) + b3_ref[...]
    out_ref[...] = jnp.maximum(z, 0.0)


def _band_matrix1(w1):
    wi = jnp.arange(38)[None, :, None]
    f = jnp.arange(1152)[None, None, :]
    x = f // 32
    c = f % 32
    d = wi - x
    dyv = jnp.arange(3)[:, None, None]
    val = w1[c, dyv, jnp.clip(d, 0, 2)]
    return jnp.where((d >= 0) & (d <= 2), val, 0.0).reshape(3 * 38, 1152)


def _band_matrix2(w2):
    fi = jnp.arange(576)[None, :, None]
    fo = jnp.arange(512)[None, None, :]
    xin, ci = fi // 32, fi % 32
    xout, co = fo // 32, fo % 32
    d = xin - xout
    dyv = jnp.arange(3)[:, None, None]
    val = w2[co, ci, dyv, jnp.clip(d, 0, 2)]
    return jnp.where((d >= 0) & (d <= 2), val, 0.0).reshape(3 * 576, 512)


def _sel_matrix(nin, nout):
    li = jnp.arange(nin)[:, None]
    lo = jnp.arange(nout)[None, :]
    return ((li // 32 == 2 * (lo // 32)) & (li % 32 == lo % 32)
            ).astype(jnp.float32)


def _fold_matrix(nin):
    li = jnp.arange(nin)[:, None]
    co = jnp.arange(32)[None, :]
    return (li % 32 == co).astype(jnp.float32)


def _cnn(img_data, p):
    img3 = img_data.reshape(_C, _IMG, _IMG)
    m1 = _band_matrix1(p['conv1_w'].reshape(32, 3, 3))
    sel1 = _sel_matrix(1152, 576)
    grid1 = _C // _NB1
    x1, sums1, sq1 = pl.pallas_call(
        _conv1_kernel,
        grid=(grid1,),
        in_specs=[
            pl.BlockSpec((_NB1, _IMG, _IMG), lambda i: (i, 0, 0)),
            pl.BlockSpec((114, 1152), lambda i: (0, 0)),
            pl.BlockSpec((1152, 576), lambda i: (0, 0)),
        ],
        out_specs=(pl.BlockSpec((_NB1, 18, 576), lambda i: (i, 0, 0)),
                   pl.BlockSpec((1, 1152), lambda i: (0, 0)),
                   pl.BlockSpec((1, 1152), lambda i: (0, 0))),
        out_shape=(jax.ShapeDtypeStruct((_C, 18, 576), jnp.float32),
                   jax.ShapeDtypeStruct((1, 1152), jnp.float32),
                   jax.ShapeDtypeStruct((1, 1152), jnp.float32)),
    )(img3, m1, sel1)

    m2 = _band_matrix2(p['conv2_w'])
    sel2 = _sel_matrix(512, 256)
    f5 = _fold_matrix(576)
    f1 = _fold_matrix(1152)
    grid2 = _C // _NB2
    x2, sums2, sq2 = pl.pallas_call(
        _conv2_kernel,
        grid=(grid2,),
        in_specs=[
            pl.BlockSpec((_NB2, 18, 576), lambda i: (i, 0, 0)),
            pl.BlockSpec((1728, 512), lambda i: (0, 0)),
            pl.BlockSpec((512, 256), lambda i: (0, 0)),
            pl.BlockSpec((576, 32), lambda i: (0, 0)),
            pl.BlockSpec((1, 1152), lambda i: (0, 0)),
            pl.BlockSpec((1, 1152), lambda i: (0, 0)),
            pl.BlockSpec((1152, 32), lambda i: (0, 0)),
            pl.BlockSpec((1, 32), lambda i: (0, 0)),
        ],
        out_specs=(pl.BlockSpec((_NB2, 8, 256), lambda i: (i, 0, 0)),
                   pl.BlockSpec((1, 512), lambda i: (0, 0)),
                   pl.BlockSpec((1, 512), lambda i: (0, 0))),
        out_shape=(jax.ShapeDtypeStruct((_C, 8, 256), jnp.float32),
                   jax.ShapeDtypeStruct((1, 512), jnp.float32),
                   jax.ShapeDtypeStruct((1, 512), jnp.float32)),
    )(x1, m2, sel2, f5, sums1, sq1, f1, p['bn1_g'].reshape(1, 32))

    hh = jnp.arange(2048) // 256
    ww = (jnp.arange(2048) // 32) % 8
    cc = jnp.arange(2048) % 32
    src = cc * 64 + hh * 8 + ww
    fcw = p['fc_w'][:, src].T
    xf = x2.reshape(_C, 2048)
    f2 = _fold_matrix(512)
    ffc = _fold_matrix(2048)
    full = lambda shape: pl.BlockSpec(shape, lambda: tuple(0 for _ in shape))
    return pl.pallas_call(
        _fc_bn3_kernel,
        in_specs=[
            full((_C, 2048)), full((2048, 128)), full((2048, 32)),
            full((1, 512)), full((1, 512)), full((512, 32)), full((1, 32)),
            full((1, 128)), full((1, 128)),
        ],
        out_specs=full((_C, 128)),
        out_shape=jax.ShapeDtypeStruct((_C, 128), jnp.float32),
    )(xf, fcw, ffc, sums2, sq2, f2, p['bn2_g'].reshape(1, 32),
      p['bn3_g'].reshape(1, 128), p['bn3_b'].reshape(1, 128))


_LB = 512


def _lstm_kernel(x_ref, wf_ref, wb_ref, bf_ref, bb_ref,
                 l1f_ref, l1b_ref, l1bias_ref, l2_ref,
                 l3f_ref, l3b_ref, l3bias_ref, out_ref):
    def step(h, c, xt, wih, whh, bias):
        g = (jnp.dot(xt, wih, preferred_element_type=jnp.float32)
             + jnp.dot(h, whh, preferred_element_type=jnp.float32) + bias)
        i = jax.nn.sigmoid(g[:, 0:128])
        f = jax.nn.sigmoid(g[:, 128:256])
        gg = jnp.tanh(g[:, 256:384])
        o = jax.nn.sigmoid(g[:, 384:512])
        c2 = f * c + i * gg
        h2 = o * jnp.tanh(c2)
        return h2, c2

    wf = wf_ref[...]
    wb = wb_ref[...]
    bf = bf_ref[...]
    bb = bb_ref[...]
    zero = jnp.zeros((_LB, 128), jnp.float32)
    hf, cf = zero, zero
    hfs = []
    for t in range(5):
        hf, cf = step(hf, cf, x_ref[:, t, :], wf[0:128], wf[128:256], bf)
        hfs.append(hf)
    hb, cb = zero, zero
    hbs = [None] * 5
    for s in range(5):
        t = 4 - s
        hb, cb = step(hb, cb, x_ref[:, t, :], wb[0:128], wb[128:256], bb)
        hbs[t] = hb

    l1f = l1f_ref[...]
    l1b = l1b_ref[...]
    l1bias = l1bias_ref[...]
    l2 = l2_ref[...]
    ss = []
    for t in range(5):
        a = jnp.tanh(jnp.dot(hfs[t], l1f, preferred_element_type=jnp.float32)
                     + jnp.dot(hbs[t], l1b, preferred_element_type=jnp.float32)
                     + l1bias)
        ss.append(jnp.sum(a * l2, axis=1, keepdims=True))
    m = ss[0]
    for t in range(1, 5):
        m = jnp.maximum(m, ss[t])
    es = [jnp.exp(s - m) for s in ss]
    z = es[0] + es[1] + es[2] + es[3] + es[4]
    yf = jnp.zeros((_LB, 128), jnp.float32)
    yb = jnp.zeros((_LB, 128), jnp.float32)
    for t in range(5):
        w = es[t] / z
        yf = yf + w * hfs[t]
        yb = yb + w * hbs[t]
    out_ref[...] = (jnp.dot(yf, l3f_ref[...], preferred_element_type=jnp.float32)
                    + jnp.dot(yb, l3b_ref[...], preferred_element_type=jnp.float32)
                    + l3bias_ref[...])


def _lstm_model(x, p):
    N = x.shape[0]
    grid = N // _LB
    wf = jnp.concatenate([p['w_ih_f'].T, p['w_hh_f'].T], axis=0)
    wb = jnp.concatenate([p['w_ih_b'].T, p['w_hh_b'].T], axis=0)
    bf = (p['b_ih_f'] + p['b_hh_f']).reshape(1, 512)
    bb = (p['b_ih_b'] + p['b_hh_b']).reshape(1, 512)
    l1t = p['lin1_w'].T
    l3t = p['lin3_w'].T
    full = lambda shape: pl.BlockSpec(shape, lambda i: tuple(0 for _ in shape))
    return pl.pallas_call(
        _lstm_kernel,
        grid=(grid,),
        in_specs=[
            pl.BlockSpec((_LB, 5, 128), lambda i: (i, 0, 0)),
            full((256, 512)), full((256, 512)),
            full((1, 512)), full((1, 512)),
            full((128, 128)), full((128, 128)), full((1, 128)), full((1, 128)),
            full((128, 128)), full((128, 128)), full((1, 128)),
        ],
        out_specs=pl.BlockSpec((_LB, 128), lambda i: (i, 0)),
        out_shape=jax.ShapeDtypeStruct((N, 128), jnp.float32),
    )(x, wf, wb, bf, bb, l1t[0:128], l1t[128:256], p['lin1_b'].reshape(1, 128),
      p['lin2_w'].reshape(1, 128), l3t[0:128], l3t[128:256],
      p['lin3_b'].reshape(1, 128))


def _log_sigmoid(x):
    return -jnp.log(1.0 + jnp.exp(-x))


def _score_kernel(u_ref, ev_ref, ec3_ref, nec_ref, nemb_ref, out_ref):
    eu = u_ref[...]
    ev = ev_ref[...]
    ec = jnp.sum(ec3_ref[...], axis=1) / float(_WORDS)
    c_score = -_log_sigmoid(jnp.clip(
        jnp.sum(eu * ec, axis=1, keepdims=True), -10.0, 10.0))
    score = -_log_sigmoid(jnp.clip(
        jnp.sum(eu * ev, axis=1, keepdims=True), -10.0, 10.0))
    neg_c = jnp.zeros_like(c_score)
    neg_s = jnp.zeros_like(c_score)
    for k in range(_NEG):
        dc = jnp.sum(nec_ref[:, k, :] * eu, axis=1, keepdims=True)
        neg_c = neg_c - _log_sigmoid(-jnp.clip(dc, -10.0, 10.0))
        ds = jnp.sum(nemb_ref[:, k, :] * eu, axis=1, keepdims=True)
        neg_s = neg_s - _log_sigmoid(-jnp.clip(ds, -10.0, 10.0))
    tot = c_score + neg_c + score + neg_s
    out_ref[...] = jnp.sum(tot, axis=0, keepdims=True) / float(_B)


def _final_score(emb_u, emb_v, echar3, negchar3, neg_embed):
    full = lambda shape: pl.BlockSpec(shape, lambda: tuple(0 for _ in shape))
    out = pl.pallas_call(
        _score_kernel,
        in_specs=[
            full((_B, 128)), full((_B, 128)), full((_B, _WORDS, 128)),
            full((_B, _NEG, 128)), full((_B, _NEG, 128)),
        ],
        out_specs=full((1, 1)),
        out_shape=jax.ShapeDtypeStruct((1, 1), jnp.float32),
    )(emb_u, emb_v, echar3, negchar3, neg_embed)
    return out.reshape(())


def kernel(pos_u, pos_v, neg_v, img_data, params, wordid2charid, noise_dist):
    p = params
    cdf, coarse = _build_cdf(noise_dist)
    uu = jax.random.uniform(jax.random.key(42), (_B, _MINI))
    noise_words = _searchsorted(uu.reshape(-1), cdf, coarse).reshape(-1)
    noise_vec = _gather_rows(p['v_table'], noise_words).reshape(_B, _MINI, 128)
    v_rows6 = _gather_rows(p['v_table'],
                           pos_v.reshape(-1).astype(jnp.int32)
                           ).reshape(_B, _WORDS, 128)
    emb_v, neg_embed, nsamp_f = _sims_topk(
        v_rows6, noise_vec,
        noise_words.reshape(_B, _MINI).astype(jnp.float32))
    neg_sample = nsamp_f[:, :_NEG].astype(jnp.int32)

    emb_u = _gather_rows(p['u_table'], pos_u.astype(jnp.int32))

    img_emb = _cnn(img_data, p)

    w2c = jnp.concatenate(
        [wordid2charid.astype(jnp.int32),
         jnp.zeros((_V, 128 - _CPW), jnp.int32)], axis=1)
    all_words = jnp.concatenate(
        [pos_v.reshape(-1).astype(jnp.int32), neg_sample.reshape(-1)])
    char_rows = _gather_rows(w2c, all_words)
    chars = char_rows[:, :_CPW].reshape(-1)
    li = _gather_rows(img_emb, chars)
    n_pos = _B * _WORDS * _CPW
    li_pos = li[:n_pos].reshape(_B * _WORDS, _CPW, 128)
    li_neg = li[n_pos:].reshape(_B * _NEG, _CPW, 128)

    echar_pos = _lstm_model(li_pos, p)
    echar_neg = _lstm_model(li_neg, p)

    return _final_score(
        emb_u, emb_v,
        echar_pos.reshape(_B, _WORDS, 128),
        echar_neg.reshape(_B, _NEG, 128),
        neg_embed)


# speedup vs baseline: 7.7847x; 7.7847x over previous
import functools

import jax
import jax.numpy as jnp
from jax import lax
from jax.experimental import pallas as pl
from jax.experimental.pallas import tpu as pltpu
from jax.experimental.pallas import tpu_sc as plsc

_V = 100000
_D = 128
_C = 2048
_B = 512
_WORDS = 6
_CPW = 5
_IMG = 38
_NEG = 5
_MINI = _B // 2

_NC = 2
_NS = 16
_NW = _NC * _NS


def _sc_gather_call(table, idx):
    B, = idx.shape
    V, Dg = table.shape
    b_per_w = B // _NW
    max_rows = max(8, (360 * 1024) // (Dg * 4))
    chunk = b_per_w
    while chunk > max_rows or chunk % 8 != 0:
        chunk //= 2
    n_chunks = b_per_w // chunk
    mesh = plsc.VectorSubcoreMesh(core_axis_name="c", subcore_axis_name="s",
                                  num_cores=_NC, num_subcores=_NS)

    @functools.partial(
        pl.kernel, mesh=mesh,
        out_type=jax.ShapeDtypeStruct((B, Dg), table.dtype),
        scratch_types=[
            pltpu.VMEM((chunk,), jnp.int32),
            pltpu.VMEM((chunk, Dg), table.dtype),
            pltpu.SemaphoreType.DMA,
        ],
    )
    def k(table_hbm, idx_hbm, out_hbm, idx_v, rows_v, sem):
        wid = lax.axis_index("s") * _NC + lax.axis_index("c")
        base = wid * b_per_w

        def body(ci, carry):
            off = base + ci * chunk
            pltpu.sync_copy(idx_hbm.at[pl.ds(off, chunk)], idx_v)
            pltpu.async_copy(table_hbm.at[idx_v], rows_v, sem).wait()
            pltpu.sync_copy(rows_v, out_hbm.at[pl.ds(off, chunk)])
            return carry

        lax.fori_loop(0, n_chunks, body, 0)

    return k(table, idx)


def _gather_rows(table, idx):
    return _sc_gather_call(table, idx)


_R = 800


def _cdf_kernel(p2_ref, cdf_ref, coarse_ref):
    p2 = p2_ref[...]
    total = jnp.sum(p2)
    p2 = p2 / total
    li = lax.broadcasted_iota(jnp.int32, (128, 128), 0)
    lj = lax.broadcasted_iota(jnp.int32, (128, 128), 1)
    lt = (li <= lj).astype(jnp.float32)
    rowcum = jnp.dot(p2, lt, preferred_element_type=jnp.float32)
    rs = jnp.dot(p2, jnp.ones((128, 1), jnp.float32),
                 preferred_element_type=jnp.float32)
    ri = lax.broadcasted_iota(jnp.int32, (_R, _R), 0)
    rj = lax.broadcasted_iota(jnp.int32, (_R, _R), 1)
    mstrict = (rj < ri).astype(jnp.float32)
    excl = jnp.dot(mstrict, rs, preferred_element_type=jnp.float32)
    cdf = rowcum + excl
    cdf_ref[...] = cdf
    e127 = (lax.broadcasted_iota(jnp.int32, (1, 128), 1) == 127
            ).astype(jnp.float32)
    coarse_ref[...] = lax.dot_general(
        e127, cdf, (((1,), (1,)), ((), ())),
        preferred_element_type=jnp.float32)


def _build_cdf(noise_dist):
    p_pad = jnp.concatenate(
        [noise_dist, jnp.zeros((_R * 128 - _V,), jnp.float32)]).reshape(_R, 128)
    return pl.pallas_call(
        _cdf_kernel,
        out_shape=(jax.ShapeDtypeStruct((_R, 128), jnp.float32),
                   jax.ShapeDtypeStruct((1, _R), jnp.float32)),
    )(p_pad)


_UBLK = 1024


def _ss_kernel(u_ref, coarse_ref, cdf_ref, out_ref):
    u = u_ref[...]
    coarse = coarse_ref[...]
    ind1 = (coarse < u).astype(jnp.float32)
    count1 = jnp.sum(ind1, axis=1, keepdims=True)
    rlane = lax.broadcasted_iota(jnp.int32, (_UBLK, _R), 1)
    onehot = (rlane == count1.astype(jnp.int32)).astype(jnp.float32)
    seg = jnp.dot(onehot, cdf_ref[...],
                  preferred_element_type=jnp.float32)
    count2 = jnp.sum((seg < u).astype(jnp.float32), axis=1, keepdims=True)
    idx = count1 * 128.0 + count2
    idx = jnp.clip(idx, 0.0, float(_V - 1))
    out_ref[...] = idx.astype(jnp.int32)


def _searchsorted(uu_flat, cdf, coarse):
    n = uu_flat.shape[0]
    grid = n // _UBLK
    return pl.pallas_call(
        _ss_kernel,
        grid=(grid,),
        in_specs=[
            pl.BlockSpec((_UBLK, 1), lambda i: (i, 0)),
            pl.BlockSpec((1, _R), lambda i: (0, 0)),
            pl.BlockSpec((_R, 128), lambda i: (0, 0)),
        ],
        out_specs=pl.BlockSpec((_UBLK, 1), lambda i: (i, 0)),
        out_shape=jax.ShapeDtypeStruct((n, 1), jnp.int32),
    )(uu_flat.reshape(n, 1), coarse, cdf)


_SB = 64


def _sims_kernel(v6_ref, nv_ref, words_ref, avg_ref, nemb_ref, nsamp_ref):
    v6 = v6_ref[...]
    avg = jnp.sum(v6, axis=1) / float(_WORDS)
    avg_ref[...] = avg
    nv = nv_ref[...]
    num = jnp.sum(nv * avg[:, None, :], axis=2)
    na = jnp.sqrt(jnp.sum(avg * avg, axis=1, keepdims=True))
    nn = jnp.sqrt(jnp.sum(nv * nv, axis=2))
    den = jnp.maximum(na * nn, 1e-8)
    sims = num / den
    words = words_ref[...]
    miota = lax.broadcasted_iota(jnp.int32, (_SB, _MINI), 1)
    cur = sims
    samp_cols = []
    for k in range(_NEG):
        m = jnp.max(cur, axis=1, keepdims=True)
        cand = jnp.where(cur >= m, miota, _MINI + 1)
        idxk = jnp.min(cand, axis=1, keepdims=True)
        onehot = (miota == idxk).astype(jnp.float32)
        nemb_ref[:, k, :] = jnp.sum(nv * onehot[:, :, None], axis=1)
        samp_cols.append(jnp.sum(words * onehot, axis=1, keepdims=True))
        cur = jnp.where(miota == idxk, -jnp.inf, cur)
    nsamp_ref[...] = jnp.concatenate(
        samp_cols + [jnp.zeros((_SB, 3), jnp.float32)], axis=1)


def _sims_topk(v_rows6, noise_vec, words_f):
    grid = _B // _SB
    return pl.pallas_call(
        _sims_kernel,
        grid=(grid,),
        in_specs=[
            pl.BlockSpec((_SB, _WORDS, 128), lambda i: (i, 0, 0)),
            pl.BlockSpec((_SB, _MINI, 128), lambda i: (i, 0, 0)),
            pl.BlockSpec((_SB, _MINI), lambda i: (i, 0)),
        ],
        out_specs=(
            pl.BlockSpec((_SB, 128), lambda i: (i, 0)),
            pl.BlockSpec((_SB, _NEG, 128), lambda i: (i, 0, 0)),
            pl.BlockSpec((_SB, 8), lambda i: (i, 0)),
        ),
        out_shape=(
            jax.ShapeDtypeStruct((_B, 128), jnp.float32),
            jax.ShapeDtypeStruct((_B, _NEG, 128), jnp.float32),
            jax.ShapeDtypeStruct((_B, 8), jnp.float32),
        ),
    )(v_rows6, noise_vec, words_f)


_NB1 = 32
_NB2 = 64


def _lane_roll32(x):
    return jnp.maximum(x, jnp.concatenate([x[:, 32:], x[:, :32]], axis=1))


def _conv1_kernel(img_ref, m1_ref, sel1_ref, out_ref, sum_ref, sq_ref):
    acc = jnp.zeros((_NB1 * 36, 1152), jnp.float32)
    for dy in range(3):
        s = img_ref[:, dy:dy + 36, :].reshape(_NB1 * 36, 38)
        acc = acc + jnp.dot(s, m1_ref[dy * 38:(dy + 1) * 38, :],
                            preferred_element_type=jnp.float32)

    @pl.when(pl.program_id(0) == 0)
    def _():
        sum_ref[...] = jnp.zeros_like(sum_ref)
        sq_ref[...] = jnp.zeros_like(sq_ref)

    sum_ref[...] += jnp.sum(acc, axis=0, keepdims=True)
    sq_ref[...] += jnp.sum(acc * acc, axis=0, keepdims=True)
    a = jnp.max(acc.reshape(_NB1, 18, 2, 1152), axis=2).reshape(_NB1 * 18, 1152)
    m = _lane_roll32(a)
    out_ref[...] = jnp.dot(m, sel1_ref[...],
                           preferred_element_type=jnp.float32
                           ).reshape(_NB1, 18, 576)


def _conv2_kernel(x_ref, m2_ref, sel2_ref, f5_ref, sum1_ref, sq1_ref,
                  f1_ref, g1_ref, out_ref, sum_ref, sq_ref):
    n1 = float(_C * 36 * 36)
    mean1 = jnp.dot(sum1_ref[...], f1_ref[...],
                    preferred_element_type=jnp.float32) / n1
    var1 = jnp.dot(sq1_ref[...], f1_ref[...],
                   preferred_element_type=jnp.float32) / n1 - mean1 * mean1
    a1 = g1_ref[...] / jnp.sqrt(var1 + 1e-5)
    a1col = lax.dot_general(f5_ref[...], a1, (((1,), (1,)), ((), ())),
                            preferred_element_type=jnp.float32)
    acc = jnp.zeros((_NB2 * 16, 512), jnp.float32)
    for dy in range(3):
        s = x_ref[:, dy:dy + 16, :].reshape(_NB2 * 16, 576)
        m2 = m2_ref[dy * 576:(dy + 1) * 576, :] * a1col
        acc = acc + jnp.dot(s, m2, preferred_element_type=jnp.float32)

    @pl.when(pl.program_id(0) == 0)
    def _():
        sum_ref[...] = jnp.zeros_like(sum_ref)
        sq_ref[...] = jnp.zeros_like(sq_ref)

    sum_ref[...] += jnp.sum(acc, axis=0, keepdims=True)
    sq_ref[...] += jnp.sum(acc * acc, axis=0, keepdims=True)
    a = jnp.max(acc.reshape(_NB2, 8, 2, 512), axis=2).reshape(_NB2 * 8, 512)
    m = _lane_roll32(a)
    out_ref[...] = jnp.dot(m, sel2_ref[...],
                           preferred_element_type=jnp.float32
                           ).reshape(_NB2, 8, 256)


def _fc_bn3_kernel(x_ref, w_ref, ffc_ref, sum2_ref, sq2_ref, f2_ref, g2_ref,
                   g3_ref, b3_ref, out_ref):
    n2 = float(_C * 16 * 16)
    mean2 = jnp.dot(sum2_ref[...], f2_ref[...],
                    preferred_element_type=jnp.float32) / n2
    var2 = jnp.dot(sq2_ref[...], f2_ref[...],
                   preferred_element_type=jnp.float32) / n2 - mean2 * mean2
    a2 = g2_ref[...] / jnp.sqrt(var2 + 1e-5)
    a2col = lax.dot_general(ffc_ref[...], a2, (((1,), (1,)), ((), ())),
                            preferred_element_type=jnp.float32)
    y = jnp.dot(x_ref[...], w_ref[...] * a2col,
                preferred_element_type=jnp.float32)
    m = jnp.mean(y, axis=0, keepdims=True)
    d = y - m
    var = jnp.mean(d * d, axis=0, keepdims=True)
    z = g3_ref[...] * d / jnp.sqrt(var + 1e-5) + b3_ref[...]
    out_ref[...] = jnp.maximum(z, 0.0)


def _band_matrix1(w1):
    x = (jnp.arange(1152) // 32)[None, :]
    wi = jnp.arange(38)[:, None]
    rows = []
    for dy in range(3):
        m = jnp.zeros((38, 1152), jnp.float32)
        for dx in range(3):
            tile = jnp.tile(w1[:, dy, dx], 36)[None, :]
            m = m + jnp.where(wi - x == dx, tile, 0.0)
        rows.append(m)
    return jnp.concatenate(rows, axis=0)


def _band_matrix2(w2):
    xin = (jnp.arange(576) // 32)[:, None]
    xout = (jnp.arange(512) // 32)[None, :]
    rows = []
    for dy in range(3):
        m = jnp.zeros((576, 512), jnp.float32)
        for dx in range(3):
            blk = jnp.tile(w2[:, :, dy, dx].T, (18, 16))
            m = m + jnp.where(xin - xout == dx, blk, 0.0)
        rows.append(m)
    return jnp.concatenate(rows, axis=0)


def _sel_matrix(nin, nout):
    li = jnp.arange(nin)[:, None]
    lo = jnp.arange(nout)[None, :]
    return ((li // 32 == 2 * (lo // 32)) & (li % 32 == lo % 32)
            ).astype(jnp.float32)


def _fold_matrix(nin):
    li = jnp.arange(nin)[:, None]
    co = jnp.arange(32)[None, :]
    return (li % 32 == co).astype(jnp.float32)


def _cnn(img_data, p):
    img3 = img_data.reshape(_C, _IMG, _IMG)
    m1 = _band_matrix1(p['conv1_w'].reshape(32, 3, 3))
    sel1 = _sel_matrix(1152, 576)
    grid1 = _C // _NB1
    x1, sums1, sq1 = pl.pallas_call(
        _conv1_kernel,
        grid=(grid1,),
        in_specs=[
            pl.BlockSpec((_NB1, _IMG, _IMG), lambda i: (i, 0, 0)),
            pl.BlockSpec((114, 1152), lambda i: (0, 0)),
            pl.BlockSpec((1152, 576), lambda i: (0, 0)),
        ],
        out_specs=(pl.BlockSpec((_NB1, 18, 576), lambda i: (i, 0, 0)),
                   pl.BlockSpec((1, 1152), lambda i: (0, 0)),
                   pl.BlockSpec((1, 1152), lambda i: (0, 0))),
        out_shape=(jax.ShapeDtypeStruct((_C, 18, 576), jnp.float32),
                   jax.ShapeDtypeStruct((1, 1152), jnp.float32),
                   jax.ShapeDtypeStruct((1, 1152), jnp.float32)),
    )(img3, m1, sel1)

    m2 = _band_matrix2(p['conv2_w'])
    sel2 = _sel_matrix(512, 256)
    f5 = _fold_matrix(576)
    f1 = _fold_matrix(1152)
    grid2 = _C // _NB2
    x2, sums2, sq2 = pl.pallas_call(
        _conv2_kernel,
        grid=(grid2,),
        in_specs=[
            pl.BlockSpec((_NB2, 18, 576), lambda i: (i, 0, 0)),
            pl.BlockSpec((1728, 512), lambda i: (0, 0)),
            pl.BlockSpec((512, 256), lambda i: (0, 0)),
            pl.BlockSpec((576, 32), lambda i: (0, 0)),
            pl.BlockSpec((1, 1152), lambda i: (0, 0)),
            pl.BlockSpec((1, 1152), lambda i: (0, 0)),
            pl.BlockSpec((1152, 32), lambda i: (0, 0)),
            pl.BlockSpec((1, 32), lambda i: (0, 0)),
        ],
        out_specs=(pl.BlockSpec((_NB2, 8, 256), lambda i: (i, 0, 0)),
                   pl.BlockSpec((1, 512), lambda i: (0, 0)),
                   pl.BlockSpec((1, 512), lambda i: (0, 0))),
        out_shape=(jax.ShapeDtypeStruct((_C, 8, 256), jnp.float32),
                   jax.ShapeDtypeStruct((1, 512), jnp.float32),
                   jax.ShapeDtypeStruct((1, 512), jnp.float32)),
    )(x1, m2, sel2, f5, sums1, sq1, f1, p['bn1_g'].reshape(1, 32))

    fcw = p['fc_w'].reshape(128, 32, 8, 8).transpose(2, 3, 1, 0).reshape(2048, 128)
    xf = x2.reshape(_C, 2048)
    f2 = _fold_matrix(512)
    ffc = _fold_matrix(2048)
    full = lambda shape: pl.BlockSpec(shape, lambda: tuple(0 for _ in shape))
    return pl.pallas_call(
        _fc_bn3_kernel,
        in_specs=[
            full((_C, 2048)), full((2048, 128)), full((2048, 32)),
            full((1, 512)), full((1, 512)), full((512, 32)), full((1, 32)),
            full((1, 128)), full((1, 128)),
        ],
        out_specs=full((_C, 128)),
        out_shape=jax.ShapeDtypeStruct((_C, 128), jnp.float32),
    )(xf, fcw, ffc, sums2, sq2, f2, p['bn2_g'].reshape(1, 32),
      p['bn3_g'].reshape(1, 128), p['bn3_b'].reshape(1, 128))


_LB = 512


def _lstm_kernel(x_ref, wf_ref, wb_ref, bf_ref, bb_ref,
                 l1f_ref, l1b_ref, l1bias_ref, l2_ref,
                 l3f_ref, l3b_ref, l3bias_ref, out_ref):
    def step(h, c, xt, wih, whh, bias):
        g = (jnp.dot(xt, wih, preferred_element_type=jnp.float32)
             + jnp.dot(h, whh, preferred_element_type=jnp.float32) + bias)
        i = jax.nn.sigmoid(g[:, 0:128])
        f = jax.nn.sigmoid(g[:, 128:256])
        gg = jnp.tanh(g[:, 256:384])
        o = jax.nn.sigmoid(g[:, 384:512])
        c2 = f * c + i * gg
        h2 = o * jnp.tanh(c2)
        return h2, c2

    wf = wf_ref[...]
    wb = wb_ref[...]
    bf = bf_ref[...]
    bb = bb_ref[...]
    zero = jnp.zeros((_LB, 128), jnp.float32)
    hf, cf = zero, zero
    hfs = []
    for t in range(5):
        hf, cf = step(hf, cf, x_ref[:, t, :], wf[0:128], wf[128:256], bf)
        hfs.append(hf)
    hb, cb = zero, zero
    hbs = [None] * 5
    for s in range(5):
        t = 4 - s
        hb, cb = step(hb, cb, x_ref[:, t, :], wb[0:128], wb[128:256], bb)
        hbs[t] = hb

    l1f = l1f_ref[...]
    l1b = l1b_ref[...]
    l1bias = l1bias_ref[...]
    l2 = l2_ref[...]
    ss = []
    for t in range(5):
        a = jnp.tanh(jnp.dot(hfs[t], l1f, preferred_element_type=jnp.float32)
                     + jnp.dot(hbs[t], l1b, preferred_element_type=jnp.float32)
                     + l1bias)
        ss.append(jnp.sum(a * l2, axis=1, keepdims=True))
    m = ss[0]
    for t in range(1, 5):
        m = jnp.maximum(m, ss[t])
    es = [jnp.exp(s - m) for s in ss]
    z = es[0] + es[1] + es[2] + es[3] + es[4]
    yf = jnp.zeros((_LB, 128), jnp.float32)
    yb = jnp.zeros((_LB, 128), jnp.float32)
    for t in range(5):
        w = es[t] / z
        yf = yf + w * hfs[t]
        yb = yb + w * hbs[t]
    out_ref[...] = (jnp.dot(yf, l3f_ref[...], preferred_element_type=jnp.float32)
                    + jnp.dot(yb, l3b_ref[...], preferred_element_type=jnp.float32)
                    + l3bias_ref[...])


def _lstm_model(x, p):
    N = x.shape[0]
    grid = N // _LB
    wf = jnp.concatenate([p['w_ih_f'].T, p['w_hh_f'].T], axis=0)
    wb = jnp.concatenate([p['w_ih_b'].T, p['w_hh_b'].T], axis=0)
    bf = (p['b_ih_f'] + p['b_hh_f']).reshape(1, 512)
    bb = (p['b_ih_b'] + p['b_hh_b']).reshape(1, 512)
    l1t = p['lin1_w'].T
    l3t = p['lin3_w'].T
    full = lambda shape: pl.BlockSpec(shape, lambda i: tuple(0 for _ in shape))
    return pl.pallas_call(
        _lstm_kernel,
        grid=(grid,),
        in_specs=[
            pl.BlockSpec((_LB, 5, 128), lambda i: (i, 0, 0)),
            full((256, 512)), full((256, 512)),
            full((1, 512)), full((1, 512)),
            full((128, 128)), full((128, 128)), full((1, 128)), full((1, 128)),
            full((128, 128)), full((128, 128)), full((1, 128)),
        ],
        out_specs=pl.BlockSpec((_LB, 128), lambda i: (i, 0)),
        out_shape=jax.ShapeDtypeStruct((N, 128), jnp.float32),
    )(x, wf, wb, bf, bb, l1t[0:128], l1t[128:256], p['lin1_b'].reshape(1, 128),
      p['lin2_w'].reshape(1, 128), l3t[0:128], l3t[128:256],
      p['lin3_b'].reshape(1, 128))


def _log_sigmoid(x):
    return -jnp.log(1.0 + jnp.exp(-x))


def _score_kernel(u_ref, ev_ref, ec3_ref, nec_ref, nemb_ref, out_ref):
    eu = u_ref[...]
    ev = ev_ref[...]
    ec = jnp.sum(ec3_ref[...], axis=1) / float(_WORDS)
    c_score = -_log_sigmoid(jnp.clip(
        jnp.sum(eu * ec, axis=1, keepdims=True), -10.0, 10.0))
    score = -_log_sigmoid(jnp.clip(
        jnp.sum(eu * ev, axis=1, keepdims=True), -10.0, 10.0))
    neg_c = jnp.zeros_like(c_score)
    neg_s = jnp.zeros_like(c_score)
    for k in range(_NEG):
        dc = jnp.sum(nec_ref[:, k, :] * eu, axis=1, keepdims=True)
        neg_c = neg_c - _log_sigmoid(-jnp.clip(dc, -10.0, 10.0))
        ds = jnp.sum(nemb_ref[:, k, :] * eu, axis=1, keepdims=True)
        neg_s = neg_s - _log_sigmoid(-jnp.clip(ds, -10.0, 10.0))
    tot = c_score + neg_c + score + neg_s
    out_ref[...] = jnp.sum(tot, axis=0, keepdims=True) / float(_B)


def _final_score(emb_u, emb_v, echar3, negchar3, neg_embed):
    full = lambda shape: pl.BlockSpec(shape, lambda: tuple(0 for _ in shape))
    out = pl.pallas_call(
        _score_kernel,
        in_specs=[
            full((_B, 128)), full((_B, 128)), full((_B, _WORDS, 128)),
            full((_B, _NEG, 128)), full((_B, _NEG, 128)),
        ],
        out_specs=full((1, 1)),
        out_shape=jax.ShapeDtypeStruct((1, 1), jnp.float32),
    )(emb_u, emb_v, echar3, negchar3, neg_embed)
    return out.reshape(())


def kernel(pos_u, pos_v, neg_v, img_data, params, wordid2charid, noise_dist):
    p = params
    cdf, coarse = _build_cdf(noise_dist)
    uu = jax.random.uniform(jax.random.key(42), (_B, _MINI))
    noise_words = _searchsorted(uu.reshape(-1), cdf, coarse).reshape(-1)
    noise_vec = _gather_rows(p['v_table'], noise_words).reshape(_B, _MINI, 128)
    v_rows6 = _gather_rows(p['v_table'],
                           pos_v.reshape(-1).astype(jnp.int32)
                           ).reshape(_B, _WORDS, 128)
    emb_v, neg_embed, nsamp_f = _sims_topk(
        v_rows6, noise_vec,
        noise_words.reshape(_B, _MINI).astype(jnp.float32))
    neg_sample = nsamp_f[:, :_NEG].astype(jnp.int32)

    emb_u = _gather_rows(p['u_table'], pos_u.astype(jnp.int32))

    img_emb = _cnn(img_data, p)

    w2c = jnp.concatenate(
        [wordid2charid.astype(jnp.int32),
         jnp.zeros((_V, 128 - _CPW), jnp.int32)], axis=1)
    all_words = jnp.concatenate(
        [pos_v.reshape(-1).astype(jnp.int32), neg_sample.reshape(-1)])
    char_rows = _gather_rows(w2c, all_words)
    chars = char_rows[:, :_CPW].reshape(-1)
    li = _gather_rows(img_emb, chars)
    n_pos = _B * _WORDS * _CPW
    li_pos = li[:n_pos].reshape(_B * _WORDS, _CPW, 128)
    li_neg = li[n_pos:].reshape(_B * _NEG, _CPW, 128)

    echar_pos = _lstm_model(li_pos, p)
    echar_neg = _lstm_model(li_neg, p)

    return _final_score(
        emb_u, emb_v,
        echar_pos.reshape(_B, _WORDS, 128),
        echar_neg.reshape(_B, _NEG, 128),
        neg_embed)


# speedup vs baseline: 8.9467x; 1.1493x over previous
import functools

import jax
import jax.numpy as jnp
from jax import lax
from jax.experimental import pallas as pl
from jax.experimental.pallas import tpu as pltpu
from jax.experimental.pallas import tpu_sc as plsc

_V = 100000
_D = 128
_C = 2048
_B = 512
_WORDS = 6
_CPW = 5
_IMG = 38
_NEG = 5
_MINI = _B // 2

_NC = 2
_NS = 16
_NW = _NC * _NS


def _sc_gather_call(table, idx):
    B, = idx.shape
    V, Dg = table.shape
    b_per_w = B // _NW
    max_rows = max(8, (360 * 1024) // (Dg * 4))
    chunk = b_per_w
    while chunk > max_rows or chunk % 8 != 0:
        chunk //= 2
    n_chunks = b_per_w // chunk
    mesh = plsc.VectorSubcoreMesh(core_axis_name="c", subcore_axis_name="s",
                                  num_cores=_NC, num_subcores=_NS)

    @functools.partial(
        pl.kernel, mesh=mesh,
        out_type=jax.ShapeDtypeStruct((B, Dg), table.dtype),
        scratch_types=[
            pltpu.VMEM((chunk,), jnp.int32),
            pltpu.VMEM((chunk, Dg), table.dtype),
            pltpu.SemaphoreType.DMA,
        ],
    )
    def k(table_hbm, idx_hbm, out_hbm, idx_v, rows_v, sem):
        wid = lax.axis_index("s") * _NC + lax.axis_index("c")
        base = wid * b_per_w

        def body(ci, carry):
            off = base + ci * chunk
            pltpu.sync_copy(idx_hbm.at[pl.ds(off, chunk)], idx_v)
            pltpu.async_copy(table_hbm.at[idx_v], rows_v, sem).wait()
            pltpu.sync_copy(rows_v, out_hbm.at[pl.ds(off, chunk)])
            return carry

        lax.fori_loop(0, n_chunks, body, 0)

    return k(table, idx)


def _gather_rows(table, idx):
    return _sc_gather_call(table, idx)


_R = 800


def _cdf_kernel(p2_ref, cdf_ref, coarse_ref):
    p2 = p2_ref[...]
    total = jnp.sum(p2)
    p2 = p2 / total
    li = lax.broadcasted_iota(jnp.int32, (128, 128), 0)
    lj = lax.broadcasted_iota(jnp.int32, (128, 128), 1)
    lt = (li <= lj).astype(jnp.float32)
    rowcum = jnp.dot(p2, lt, preferred_element_type=jnp.float32)
    rs = jnp.dot(p2, jnp.ones((128, 1), jnp.float32),
                 preferred_element_type=jnp.float32)
    ri = lax.broadcasted_iota(jnp.int32, (_R, _R), 0)
    rj = lax.broadcasted_iota(jnp.int32, (_R, _R), 1)
    mstrict = (rj < ri).astype(jnp.float32)
    excl = jnp.dot(mstrict, rs, preferred_element_type=jnp.float32)
    cdf = rowcum + excl
    cdf_ref[...] = cdf
    e127 = (lax.broadcasted_iota(jnp.int32, (1, 128), 1) == 127
            ).astype(jnp.float32)
    coarse_ref[...] = lax.dot_general(
        e127, cdf, (((1,), (1,)), ((), ())),
        preferred_element_type=jnp.float32)


def _build_cdf(noise_dist):
    p_pad = jnp.concatenate(
        [noise_dist, jnp.zeros((_R * 128 - _V,), jnp.float32)]).reshape(_R, 128)
    return pl.pallas_call(
        _cdf_kernel,
        out_shape=(jax.ShapeDtypeStruct((_R, 128), jnp.float32),
                   jax.ShapeDtypeStruct((1, _R), jnp.float32)),
    )(p_pad)


_UBLK = 1024


def _ss_kernel(u_ref, coarse_ref, cdf_ref, out_ref):
    u = u_ref[...]
    coarse = coarse_ref[...]
    ind1 = (coarse < u).astype(jnp.float32)
    count1 = jnp.sum(ind1, axis=1, keepdims=True)
    rlane = lax.broadcasted_iota(jnp.int32, (_UBLK, _R), 1)
    onehot = (rlane == count1.astype(jnp.int32)).astype(jnp.float32)
    seg = jnp.dot(onehot, cdf_ref[...],
                  preferred_element_type=jnp.float32)
    count2 = jnp.sum((seg < u).astype(jnp.float32), axis=1, keepdims=True)
    idx = count1 * 128.0 + count2
    idx = jnp.clip(idx, 0.0, float(_V - 1))
    out_ref[...] = idx.astype(jnp.int32)


def _searchsorted(uu_flat, cdf, coarse):
    n = uu_flat.shape[0]
    grid = n // _UBLK
    return pl.pallas_call(
        _ss_kernel,
        grid=(grid,),
        in_specs=[
            pl.BlockSpec((_UBLK, 1), lambda i: (i, 0)),
            pl.BlockSpec((1, _R), lambda i: (0, 0)),
            pl.BlockSpec((_R, 128), lambda i: (0, 0)),
        ],
        out_specs=pl.BlockSpec((_UBLK, 1), lambda i: (i, 0)),
        out_shape=jax.ShapeDtypeStruct((n, 1), jnp.int32),
    )(uu_flat.reshape(n, 1), coarse, cdf)


_SB = 64


def _sims_kernel(v6_ref, nv_ref, words_ref, avg_ref, nemb_ref, nsamp_ref):
    v6 = v6_ref[...]
    avg = jnp.sum(v6, axis=1) / float(_WORDS)
    avg_ref[...] = avg
    nv = nv_ref[...]
    num = jnp.sum(nv * avg[:, None, :], axis=2)
    na = jnp.sqrt(jnp.sum(avg * avg, axis=1, keepdims=True))
    nn = jnp.sqrt(jnp.sum(nv * nv, axis=2))
    den = jnp.maximum(na * nn, 1e-8)
    sims = num / den
    words = words_ref[...]
    miota = lax.broadcasted_iota(jnp.int32, (_SB, _MINI), 1)
    cur = sims
    samp_cols = []
    for k in range(_NEG):
        m = jnp.max(cur, axis=1, keepdims=True)
        cand = jnp.where(cur >= m, miota, _MINI + 1)
        idxk = jnp.min(cand, axis=1, keepdims=True)
        onehot = (miota == idxk).astype(jnp.float32)
        nemb_ref[:, k, :] = jnp.sum(nv * onehot[:, :, None], axis=1)
        samp_cols.append(jnp.sum(words * onehot, axis=1, keepdims=True))
        cur = jnp.where(miota == idxk, -jnp.inf, cur)
    nsamp_ref[...] = jnp.concatenate(
        samp_cols + [jnp.zeros((_SB, 3), jnp.float32)], axis=1)


def _sims_topk(v_rows6, noise_vec, words_f):
    grid = _B // _SB
    return pl.pallas_call(
        _sims_kernel,
        grid=(grid,),
        in_specs=[
            pl.BlockSpec((_SB, _WORDS, 128), lambda i: (i, 0, 0)),
            pl.BlockSpec((_SB, _MINI, 128), lambda i: (i, 0, 0)),
            pl.BlockSpec((_SB, _MINI), lambda i: (i, 0)),
        ],
        out_specs=(
            pl.BlockSpec((_SB, 128), lambda i: (i, 0)),
            pl.BlockSpec((_SB, _NEG, 128), lambda i: (i, 0, 0)),
            pl.BlockSpec((_SB, 8), lambda i: (i, 0)),
        ),
        out_shape=(
            jax.ShapeDtypeStruct((_B, 128), jnp.float32),
            jax.ShapeDtypeStruct((_B, _NEG, 128), jnp.float32),
            jax.ShapeDtypeStruct((_B, 8), jnp.float32),
        ),
    )(v_rows6, noise_vec, words_f)


_NB1 = 32
_NB2 = 64


def _lane_roll32(x):
    return jnp.maximum(x, jnp.concatenate([x[:, 32:], x[:, :32]], axis=1))


def _conv1_kernel(imge_ref, imgo_ref, m1_ref, sel1_ref, out_ref, sum_ref, sq_ref):
    def mm(ref, lo, dy):
        s = ref[:, lo:lo + 18, :].reshape(_NB1 * 18, 38)
        return jnp.dot(s, m1_ref[dy * 38:(dy + 1) * 38, :],
                       preferred_element_type=jnp.float32)

    acc_a = mm(imge_ref, 0, 0) + mm(imgo_ref, 0, 1) + mm(imge_ref, 1, 2)
    acc_b = mm(imgo_ref, 0, 0) + mm(imge_ref, 1, 1) + mm(imgo_ref, 1, 2)

    @pl.when(pl.program_id(0) == 0)
    def _():
        sum_ref[...] = jnp.zeros_like(sum_ref)
        sq_ref[...] = jnp.zeros_like(sq_ref)

    sum_ref[...] += (jnp.sum(acc_a, axis=0, keepdims=True)
                     + jnp.sum(acc_b, axis=0, keepdims=True))
    sq_ref[...] += (jnp.sum(acc_a * acc_a, axis=0, keepdims=True)
                    + jnp.sum(acc_b * acc_b, axis=0, keepdims=True))
    m = _lane_roll32(jnp.maximum(acc_a, acc_b))
    out_ref[...] = jnp.dot(m, sel1_ref[...],
                           preferred_element_type=jnp.float32
                           ).reshape(_NB1, 18, 576)


def _conv2_kernel(xe_ref, xo_ref, m2_ref, sel2_ref, f5_ref, sum1_ref, sq1_ref,
                  f1_ref, g1_ref, out_ref, sum_ref, sq_ref):
    n1 = float(_C * 36 * 36)
    mean1 = jnp.dot(sum1_ref[...], f1_ref[...],
                    preferred_element_type=jnp.float32) / n1
    var1 = jnp.dot(sq1_ref[...], f1_ref[...],
                   preferred_element_type=jnp.float32) / n1 - mean1 * mean1
    a1 = g1_ref[...] / jnp.sqrt(var1 + 1e-5)
    a1col = lax.dot_general(f5_ref[...], a1, (((1,), (1,)), ((), ())),
                            preferred_element_type=jnp.float32)

    def mm(ref, lo, dy):
        s = ref[:, lo:lo + 8, :].reshape(_NB2 * 8, 576)
        m2 = m2_ref[dy * 576:(dy + 1) * 576, :] * a1col
        return jnp.dot(s, m2, preferred_element_type=jnp.float32)

    acc_a = mm(xe_ref, 0, 0) + mm(xo_ref, 0, 1) + mm(xe_ref, 1, 2)
    acc_b = mm(xo_ref, 0, 0) + mm(xe_ref, 1, 1) + mm(xo_ref, 1, 2)

    @pl.when(pl.program_id(0) == 0)
    def _():
        sum_ref[...] = jnp.zeros_like(sum_ref)
        sq_ref[...] = jnp.zeros_like(sq_ref)

    sum_ref[...] += (jnp.sum(acc_a, axis=0, keepdims=True)
                     + jnp.sum(acc_b, axis=0, keepdims=True))
    sq_ref[...] += (jnp.sum(acc_a * acc_a, axis=0, keepdims=True)
                    + jnp.sum(acc_b * acc_b, axis=0, keepdims=True))
    m = _lane_roll32(jnp.maximum(acc_a, acc_b))
    out_ref[...] = jnp.dot(m, sel2_ref[...],
                           preferred_element_type=jnp.float32
                           ).reshape(_NB2, 8, 256)


def _fc_bn3_kernel(x_ref, w_ref, ffc_ref, sum2_ref, sq2_ref, f2_ref, g2_ref,
                   g3_ref, b3_ref, out_ref):
    n2 = float(_C * 16 * 16)
    mean2 = jnp.dot(sum2_ref[...], f2_ref[...],
                    preferred_element_type=jnp.float32) / n2
    var2 = jnp.dot(sq2_ref[...], f2_ref[...],
                   preferred_element_type=jnp.float32) / n2 - mean2 * mean2
    a2 = g2_ref[...] / jnp.sqrt(var2 + 1e-5)
    a2col = lax.dot_general(ffc_ref[...], a2, (((1,), (1,)), ((), ())),
                            preferred_element_type=jnp.float32)
    y = jnp.dot(x_ref[...], w_ref[...] * a2col,
                preferred_element_type=jnp.float32)
    m = jnp.mean(y, axis=0, keepdims=True)
    d = y - m
    var = jnp.mean(d * d, axis=0, keepdims=True)
    z = g3_ref[...] * d / jnp.sqrt(var + 1e-5) + b3_ref[...]
    out_ref[...] = jnp.maximum(z, 0.0)


def _band_matrix1(w1):
    x = (jnp.arange(1152) // 32)[None, :]
    wi = jnp.arange(38)[:, None]
    rows = []
    for dy in range(3):
        m = jnp.zeros((38, 1152), jnp.float32)
        for dx in range(3):
            tile = jnp.tile(w1[:, dy, dx], 36)[None, :]
            m = m + jnp.where(wi - x == dx, tile, 0.0)
        rows.append(m)
    return jnp.concatenate(rows, axis=0)


def _band_matrix2(w2):
    xin = (jnp.arange(576) // 32)[:, None]
    xout = (jnp.arange(512) // 32)[None, :]
    rows = []
    for dy in range(3):
        m = jnp.zeros((576, 512), jnp.float32)
        for dx in range(3):
            blk = jnp.tile(w2[:, :, dy, dx].T, (18, 16))
            m = m + jnp.where(xin - xout == dx, blk, 0.0)
        rows.append(m)
    return jnp.concatenate(rows, axis=0)


def _sel_matrix(nin, nout):
    li = jnp.arange(nin)[:, None]
    lo = jnp.arange(nout)[None, :]
    return ((li // 32 == 2 * (lo // 32)) & (li % 32 == lo % 32)
            ).astype(jnp.float32)


def _fold_matrix(nin):
    li = jnp.arange(nin)[:, None]
    co = jnp.arange(32)[None, :]
    return (li % 32 == co).astype(jnp.float32)


def _cnn(img_data, p):
    img4 = img_data.reshape(_C, 19, 2, _IMG)
    imge, imgo = img4[:, :, 0, :], img4[:, :, 1, :]
    m1 = _band_matrix1(p['conv1_w'].reshape(32, 3, 3))
    sel1 = _sel_matrix(1152, 576)
    grid1 = _C // _NB1
    x1, sums1, sq1 = pl.pallas_call(
        _conv1_kernel,
        grid=(grid1,),
        in_specs=[
            pl.BlockSpec((_NB1, 19, _IMG), lambda i: (i, 0, 0)),
            pl.BlockSpec((_NB1, 19, _IMG), lambda i: (i, 0, 0)),
            pl.BlockSpec((114, 1152), lambda i: (0, 0)),
            pl.BlockSpec((1152, 576), lambda i: (0, 0)),
        ],
        out_specs=(pl.BlockSpec((_NB1, 18, 576), lambda i: (i, 0, 0)),
                   pl.BlockSpec((1, 1152), lambda i: (0, 0)),
                   pl.BlockSpec((1, 1152), lambda i: (0, 0))),
        out_shape=(jax.ShapeDtypeStruct((_C, 18, 576), jnp.float32),
                   jax.ShapeDtypeStruct((1, 1152), jnp.float32),
                   jax.ShapeDtypeStruct((1, 1152), jnp.float32)),
    )(imge, imgo, m1, sel1)
    x1r = x1.reshape(_C, 9, 2, 576)
    x1e, x1o = x1r[:, :, 0, :], x1r[:, :, 1, :]

    m2 = _band_matrix2(p['conv2_w'])
    sel2 = _sel_matrix(512, 256)
    f5 = _fold_matrix(576)
    f1 = _fold_matrix(1152)
    grid2 = _C // _NB2
    x2, sums2, sq2 = pl.pallas_call(
        _conv2_kernel,
        grid=(grid2,),
        in_specs=[
            pl.BlockSpec((_NB2, 9, 576), lambda i: (i, 0, 0)),
            pl.BlockSpec((_NB2, 9, 576), lambda i: (i, 0, 0)),
            pl.BlockSpec((1728, 512), lambda i: (0, 0)),
            pl.BlockSpec((512, 256), lambda i: (0, 0)),
            pl.BlockSpec((576, 32), lambda i: (0, 0)),
            pl.BlockSpec((1, 1152), lambda i: (0, 0)),
            pl.BlockSpec((1, 1152), lambda i: (0, 0)),
            pl.BlockSpec((1152, 32), lambda i: (0, 0)),
            pl.BlockSpec((1, 32), lambda i: (0, 0)),
        ],
        out_specs=(pl.BlockSpec((_NB2, 8, 256), lambda i: (i, 0, 0)),
                   pl.BlockSpec((1, 512), lambda i: (0, 0)),
                   pl.BlockSpec((1, 512), lambda i: (0, 0))),
        out_shape=(jax.ShapeDtypeStruct((_C, 8, 256), jnp.float32),
                   jax.ShapeDtypeStruct((1, 512), jnp.float32),
                   jax.ShapeDtypeStruct((1, 512), jnp.float32)),
    )(x1e, x1o, m2, sel2, f5, sums1, sq1, f1, p['bn1_g'].reshape(1, 32))

    fcw = p['fc_w'].reshape(128, 32, 8, 8).transpose(2, 3, 1, 0).reshape(2048, 128)
    xf = x2.reshape(_C, 2048)
    f2 = _fold_matrix(512)
    ffc = _fold_matrix(2048)
    full = lambda shape: pl.BlockSpec(shape, lambda: tuple(0 for _ in shape))
    return pl.pallas_call(
        _fc_bn3_kernel,
        in_specs=[
            full((_C, 2048)), full((2048, 128)), full((2048, 32)),
            full((1, 512)), full((1, 512)), full((512, 32)), full((1, 32)),
            full((1, 128)), full((1, 128)),
        ],
        out_specs=full((_C, 128)),
        out_shape=jax.ShapeDtypeStruct((_C, 128), jnp.float32),
    )(xf, fcw, ffc, sums2, sq2, f2, p['bn2_g'].reshape(1, 32),
      p['bn3_g'].reshape(1, 128), p['bn3_b'].reshape(1, 128))


_LB = 512


def _lstm_kernel(x_ref, wf_ref, wb_ref, bf_ref, bb_ref,
                 l1f_ref, l1b_ref, l1bias_ref, l2_ref,
                 l3f_ref, l3b_ref, l3bias_ref, out_ref):
    def step(h, c, xt, wih, whh, bias):
        g = (jnp.dot(xt, wih, preferred_element_type=jnp.float32)
             + jnp.dot(h, whh, preferred_element_type=jnp.float32) + bias)
        i = jax.nn.sigmoid(g[:, 0:128])
        f = jax.nn.sigmoid(g[:, 128:256])
        gg = jnp.tanh(g[:, 256:384])
        o = jax.nn.sigmoid(g[:, 384:512])
        c2 = f * c + i * gg
        h2 = o * jnp.tanh(c2)
        return h2, c2

    wf = wf_ref[...]
    wb = wb_ref[...]
    bf = bf_ref[...]
    bb = bb_ref[...]
    zero = jnp.zeros((_LB, 128), jnp.float32)
    hf, cf = zero, zero
    hfs = []
    for t in range(5):
        hf, cf = step(hf, cf, x_ref[:, t, :], wf[0:128], wf[128:256], bf)
        hfs.append(hf)
    hb, cb = zero, zero
    hbs = [None] * 5
    for s in range(5):
        t = 4 - s
        hb, cb = step(hb, cb, x_ref[:, t, :], wb[0:128], wb[128:256], bb)
        hbs[t] = hb

    l1f = l1f_ref[...]
    l1b = l1b_ref[...]
    l1bias = l1bias_ref[...]
    l2 = l2_ref[...]
    ss = []
    for t in range(5):
        a = jnp.tanh(jnp.dot(hfs[t], l1f, preferred_element_type=jnp.float32)
                     + jnp.dot(hbs[t], l1b, preferred_element_type=jnp.float32)
                     + l1bias)
        ss.append(jnp.sum(a * l2, axis=1, keepdims=True))
    m = ss[0]
    for t in range(1, 5):
        m = jnp.maximum(m, ss[t])
    es = [jnp.exp(s - m) for s in ss]
    z = es[0] + es[1] + es[2] + es[3] + es[4]
    yf = jnp.zeros((_LB, 128), jnp.float32)
    yb = jnp.zeros((_LB, 128), jnp.float32)
    for t in range(5):
        w = es[t] / z
        yf = yf + w * hfs[t]
        yb = yb + w * hbs[t]
    out_ref[...] = (jnp.dot(yf, l3f_ref[...], preferred_element_type=jnp.float32)
                    + jnp.dot(yb, l3b_ref[...], preferred_element_type=jnp.float32)
                    + l3bias_ref[...])


def _lstm_model(x, p):
    N = x.shape[0]
    grid = N // _LB
    wf = jnp.concatenate([p['w_ih_f'].T, p['w_hh_f'].T], axis=0)
    wb = jnp.concatenate([p['w_ih_b'].T, p['w_hh_b'].T], axis=0)
    bf = (p['b_ih_f'] + p['b_hh_f']).reshape(1, 512)
    bb = (p['b_ih_b'] + p['b_hh_b']).reshape(1, 512)
    l1t = p['lin1_w'].T
    l3t = p['lin3_w'].T
    full = lambda shape: pl.BlockSpec(shape, lambda i: tuple(0 for _ in shape))
    return pl.pallas_call(
        _lstm_kernel,
        grid=(grid,),
        in_specs=[
            pl.BlockSpec((_LB, 5, 128), lambda i: (i, 0, 0)),
            full((256, 512)), full((256, 512)),
            full((1, 512)), full((1, 512)),
            full((128, 128)), full((128, 128)), full((1, 128)), full((1, 128)),
            full((128, 128)), full((128, 128)), full((1, 128)),
        ],
        out_specs=pl.BlockSpec((_LB, 128), lambda i: (i, 0)),
        out_shape=jax.ShapeDtypeStruct((N, 128), jnp.float32),
    )(x, wf, wb, bf, bb, l1t[0:128], l1t[128:256], p['lin1_b'].reshape(1, 128),
      p['lin2_w'].reshape(1, 128), l3t[0:128], l3t[128:256],
      p['lin3_b'].reshape(1, 128))


def _log_sigmoid(x):
    return -jnp.log(1.0 + jnp.exp(-x))


def _score_kernel(u_ref, ev_ref, ec3_ref, nec_ref, nemb_ref, out_ref):
    eu = u_ref[...]
    ev = ev_ref[...]
    ec = jnp.sum(ec3_ref[...], axis=1) / float(_WORDS)
    c_score = -_log_sigmoid(jnp.clip(
        jnp.sum(eu * ec, axis=1, keepdims=True), -10.0, 10.0))
    score = -_log_sigmoid(jnp.clip(
        jnp.sum(eu * ev, axis=1, keepdims=True), -10.0, 10.0))
    neg_c = jnp.zeros_like(c_score)
    neg_s = jnp.zeros_like(c_score)
    for k in range(_NEG):
        dc = jnp.sum(nec_ref[:, k, :] * eu, axis=1, keepdims=True)
        neg_c = neg_c - _log_sigmoid(-jnp.clip(dc, -10.0, 10.0))
        ds = jnp.sum(nemb_ref[:, k, :] * eu, axis=1, keepdims=True)
        neg_s = neg_s - _log_sigmoid(-jnp.clip(ds, -10.0, 10.0))
    tot = c_score + neg_c + score + neg_s
    out_ref[...] = jnp.sum(tot, axis=0, keepdims=True) / float(_B)


def _final_score(emb_u, emb_v, echar3, negchar3, neg_embed):
    full = lambda shape: pl.BlockSpec(shape, lambda: tuple(0 for _ in shape))
    out = pl.pallas_call(
        _score_kernel,
        in_specs=[
            full((_B, 128)), full((_B, 128)), full((_B, _WORDS, 128)),
            full((_B, _NEG, 128)), full((_B, _NEG, 128)),
        ],
        out_specs=full((1, 1)),
        out_shape=jax.ShapeDtypeStruct((1, 1), jnp.float32),
    )(emb_u, emb_v, echar3, negchar3, neg_embed)
    return out.reshape(())


def kernel(pos_u, pos_v, neg_v, img_data, params, wordid2charid, noise_dist):
    p = params
    cdf, coarse = _build_cdf(noise_dist)
    uu = jax.random.uniform(jax.random.key(42), (_B, _MINI))
    noise_words = _searchsorted(uu.reshape(-1), cdf, coarse).reshape(-1)
    noise_vec = _gather_rows(p['v_table'], noise_words).reshape(_B, _MINI, 128)
    v_rows6 = _gather_rows(p['v_table'],
                           pos_v.reshape(-1).astype(jnp.int32)
                           ).reshape(_B, _WORDS, 128)
    emb_v, neg_embed, nsamp_f = _sims_topk(
        v_rows6, noise_vec,
        noise_words.reshape(_B, _MINI).astype(jnp.float32))
    neg_sample = nsamp_f[:, :_NEG].astype(jnp.int32)

    emb_u = _gather_rows(p['u_table'], pos_u.astype(jnp.int32))

    img_emb = _cnn(img_data, p)

    w2c = jnp.concatenate(
        [wordid2charid.astype(jnp.int32),
         jnp.zeros((_V, 128 - _CPW), jnp.int32)], axis=1)
    all_words = jnp.concatenate(
        [pos_v.reshape(-1).astype(jnp.int32), neg_sample.reshape(-1)])
    char_rows = _gather_rows(w2c, all_words)
    chars = char_rows[:, :_CPW].reshape(-1)
    li = _gather_rows(img_emb, chars)
    n_pos = _B * _WORDS * _CPW
    li_pos = li[:n_pos].reshape(_B * _WORDS, _CPW, 128)
    li_neg = li[n_pos:].reshape(_B * _NEG, _CPW, 128)

    echar_pos = _lstm_model(li_pos, p)
    echar_neg = _lstm_model(li_neg, p)

    return _final_score(
        emb_u, emb_v,
        echar_pos.reshape(_B, _WORDS, 128),
        echar_neg.reshape(_B, _NEG, 128),
        neg_embed)
